# baseline (speedup 1.0000x reference)
import functools

import jax
import jax.numpy as jnp
from jax import lax
from jax.experimental import pallas as pl
from jax.experimental.pallas import tpu as pltpu
from jax.experimental.pallas import tpu_sc as plsc

_N = 10000
_D = 128
_TOTAL = _N * _D
_NW = 32
_CPW = 8
_CHUNK = _TOTAL // (_NW * _CPW)
_LANES = 16
_UNROLL = 5


def _make_sc_add():
    mesh = plsc.VectorSubcoreMesh(core_axis_name="c", subcore_axis_name="s")

    @functools.partial(
        pl.kernel,
        mesh=mesh,
        out_type=jax.ShapeDtypeStruct((_TOTAL,), jnp.float32),
        scratch_types=(
            [pltpu.VMEM((_CHUNK,), jnp.float32) for _ in range(2 * _CPW)]
            + [pltpu.SemaphoreType.DMA for _ in range(3 * _CPW)]
        ),
    )
    def sc_add(x_hbm, pos_hbm, out_hbm, *scratch):
        bufx = scratch[0:_CPW]
        bufp = scratch[_CPW:2 * _CPW]
        sx = scratch[2 * _CPW:3 * _CPW]
        sp = scratch[3 * _CPW:4 * _CPW]
        so = scratch[4 * _CPW:5 * _CPW]
        wid = lax.axis_index("s") * 2 + lax.axis_index("c")

        def start_in(t):
            base = (wid * _CPW + t) * _CHUNK
            hx = pltpu.async_copy(x_hbm.at[pl.ds(base, _CHUNK)], bufx[t], sx[t])
            hp = pltpu.async_copy(pos_hbm.at[pl.ds(base, _CHUNK)], bufp[t], sp[t])
            return hx, hp

        def start_out(t):
            base = (wid * _CPW + t) * _CHUNK
            return pltpu.async_copy(bufx[t], out_hbm.at[pl.ds(base, _CHUNK)], so[t])

        def compute(t):
            xv, pv = bufx[t], bufp[t]

            @plsc.parallel_loop(0, _CHUNK, step=_LANES, unroll=_UNROLL)
            def body(i):
                sl = pl.ds(i, _LANES)
                plsc.addupdate(xv.at[sl], pv[sl])

        in_h = [start_in(t) for t in range(_CPW)]
        out_h = []
        for t in range(_CPW):
            hx, hp = in_h[t]
            hx.wait()
            hp.wait()
            compute(t)
            out_h.append(start_out(t))
        for h in out_h:
            h.wait()

    return sc_add


_sc_add = _make_sc_add()


def kernel(x, edge_index, pos_embedding):
    n, d = x.shape
    out_flat = _sc_add(x.reshape(-1), pos_embedding.reshape(-1))
    return out_flat.reshape(n, d)


# speedup vs baseline: 1.0181x; 1.0181x over previous
import functools

import jax
import jax.numpy as jnp
from jax import lax
from jax.experimental import pallas as pl
from jax.experimental.pallas import tpu as pltpu
from jax.experimental.pallas import tpu_sc as plsc

_N = 10000
_D = 128
_TOTAL = _N * _D
_NW = 32
_CPW = 4
_CHUNK = _TOTAL // (_NW * _CPW)
_LANES = 16
_UNROLL = 5


def _make_sc_add():
    mesh = plsc.VectorSubcoreMesh(core_axis_name="c", subcore_axis_name="s")

    @functools.partial(
        pl.kernel,
        mesh=mesh,
        out_type=jax.ShapeDtypeStruct((_TOTAL,), jnp.float32),
        scratch_types=(
            [pltpu.VMEM((_CHUNK,), jnp.float32) for _ in range(2 * _CPW)]
            + [pltpu.SemaphoreType.DMA for _ in range(3 * _CPW)]
        ),
    )
    def sc_add(x_hbm, pos_hbm, out_hbm, *scratch):
        bufx = scratch[0:_CPW]
        bufp = scratch[_CPW:2 * _CPW]
        sx = scratch[2 * _CPW:3 * _CPW]
        sp = scratch[3 * _CPW:4 * _CPW]
        so = scratch[4 * _CPW:5 * _CPW]
        wid = lax.axis_index("s") * 2 + lax.axis_index("c")

        def start_in(t):
            base = (wid * _CPW + t) * _CHUNK
            hx = pltpu.async_copy(x_hbm.at[pl.ds(base, _CHUNK)], bufx[t], sx[t])
            hp = pltpu.async_copy(pos_hbm.at[pl.ds(base, _CHUNK)], bufp[t], sp[t])
            return hx, hp

        def start_out(t):
            base = (wid * _CPW + t) * _CHUNK
            return pltpu.async_copy(bufx[t], out_hbm.at[pl.ds(base, _CHUNK)], so[t])

        def compute(t):
            xv, pv = bufx[t], bufp[t]

            @plsc.parallel_loop(0, _CHUNK, step=_LANES, unroll=_UNROLL)
            def body(i):
                sl = pl.ds(i, _LANES)
                plsc.addupdate(xv.at[sl], pv[sl])

        in_h = [start_in(t) for t in range(_CPW)]
        out_h = []
        for t in range(_CPW):
            hx, hp = in_h[t]
            hx.wait()
            hp.wait()
            compute(t)
            out_h.append(start_out(t))
        for h in out_h:
            h.wait()

    return sc_add


_sc_add = _make_sc_add()


def kernel(x, edge_index, pos_embedding):
    n, d = x.shape
    out_flat = _sc_add(x.reshape(-1), pos_embedding.reshape(-1))
    return out_flat.reshape(n, d)
